# baseline (speedup 1.0000x reference)
import functools

import jax
import jax.numpy as jnp
from jax import lax
from jax.experimental import pallas as pl
from jax.experimental.pallas import tpu as pltpu
from jax.experimental.pallas import tpu_sc as plsc

INT_FEATURES = 26
FLOAT_FEATURES = 13
VOCAB = 100000
EMBED = 16
HIDDEN = 128

NUM_WORKERS = 32
CHUNK = 128


def _sc_gather(table16, table0, idx3d, rows_per):
    nchunk = rows_per // CHUNK
    rows_total = NUM_WORKERS * rows_per
    mesh = plsc.VectorSubcoreMesh(core_axis_name="c", subcore_axis_name="s")

    @functools.partial(
        pl.kernel,
        out_type=[
            jax.ShapeDtypeStruct((rows_total, EMBED), jnp.float32),
            jax.ShapeDtypeStruct((rows_total,), jnp.float32),
        ],
        mesh=mesh,
        scratch_types=[
            pltpu.VMEM((CHUNK,), jnp.int32),
            pltpu.VMEM((rows_per, EMBED), jnp.float32),
            pltpu.VMEM((rows_per,), jnp.float32),
            pltpu.SemaphoreType.DMA,
            pltpu.SemaphoreType.DMA,
        ],
        compiler_params=pltpu.CompilerParams(use_tc_tiling_on_sc=False),
    )
    def gather_kernel(t16_hbm, t0_hbm, idx_hbm, out16_hbm, out0_hbm,
                      idx1_v, rows_v, row0_v, sem_a, sem_b):
        wid = lax.axis_index("s") * 2 + lax.axis_index("c")
        base = wid * rows_per

        def gather_chunk(j, carry):
            pltpu.sync_copy(idx_hbm.at[wid, j], idx1_v)
            cp_a = pltpu.make_async_copy(
                t16_hbm.at[idx1_v],
                rows_v.at[pl.ds(j * CHUNK, CHUNK), :],
                sem_a,
            )
            cp_b = pltpu.make_async_copy(
                t0_hbm.at[idx1_v],
                row0_v.at[pl.ds(j * CHUNK, CHUNK)],
                sem_b,
            )
            cp_a.start()
            cp_b.start()
            cp_a.wait()
            cp_b.wait()
            return carry

        lax.fori_loop(0, nchunk, gather_chunk, 0)
        pltpu.sync_copy(rows_v, out16_hbm.at[pl.ds(base, rows_per), :])
        pltpu.sync_copy(row0_v, out0_hbm.at[pl.ds(base, rows_per)])

    return gather_kernel(table16, table0, idx3d)


def _tc_body(e_ref, f0_ref, x2_ref, g_ref, s_ref, vf_ref, vf2_ref, w1_ref,
             w3_ref, c2_ref, b3_ref, o_ref):
    E = e_ref[...]
    X2p = x2_ref[...]
    M = jnp.dot(E, g_ref[...], preferred_element_type=jnp.float32)
    M2 = jnp.dot(E * E, s_ref[...], preferred_element_type=jnp.float32)
    sum_f = jnp.dot(X2p, vf_ref[...], preferred_element_type=jnp.float32)
    sumsq_f = jnp.dot(X2p * X2p, vf2_ref[...],
                      preferred_element_type=jnp.float32)
    lin_f = jnp.dot(X2p, w1_ref[...], preferred_element_type=jnp.float32)

    H = M[:, :HIDDEN] + c2_ref[...]
    xv_sum = M[:, HIDDEN:HIDDEN + EMBED] + sum_f
    s0 = jnp.sum(f0_ref[...], axis=1, keepdims=True)
    xv_sq = M2 + sumsq_f

    inter = 0.5 * jnp.sum(xv_sum * xv_sum - xv_sq, axis=1, keepdims=True)
    y_fm = lin_f + s0 + inter

    h1 = H / (1.0 + jnp.exp(-H))
    d = jnp.dot(h1, w3_ref[...], preferred_element_type=jnp.float32) \
        + b3_ref[...]
    y_dnn = d / (1.0 + jnp.exp(-d))
    o_ref[...] = y_fm + y_dnn


def _tc_dense(E, F0, X2p, G, S, Vfp, Vf2p, W1p, W3, c2, b3):
    B = E.shape[0]
    BLK = 512
    grid = (B // BLK,)
    D = INT_FEATURES * EMBED
    return pl.pallas_call(
        _tc_body,
        grid=grid,
        in_specs=[
            pl.BlockSpec((BLK, D), lambda i: (i, 0)),
            pl.BlockSpec((BLK, INT_FEATURES), lambda i: (i, 0)),
            pl.BlockSpec((BLK, 16), lambda i: (i, 0)),
            pl.BlockSpec((D, 256), lambda i: (0, 0)),
            pl.BlockSpec((D, EMBED), lambda i: (0, 0)),
            pl.BlockSpec((16, EMBED), lambda i: (0, 0)),
            pl.BlockSpec((16, EMBED), lambda i: (0, 0)),
            pl.BlockSpec((16, 1), lambda i: (0, 0)),
            pl.BlockSpec((HIDDEN, 1), lambda i: (0, 0)),
            pl.BlockSpec((1, HIDDEN), lambda i: (0, 0)),
            pl.BlockSpec((1, 1), lambda i: (0, 0)),
        ],
        out_specs=pl.BlockSpec((BLK, 1), lambda i: (i, 0)),
        out_shape=jax.ShapeDtypeStruct((B, 1), jnp.float32),
        compiler_params=pltpu.CompilerParams(
            dimension_semantics=("arbitrary",)),
    )(E, F0, X2p, G, S, Vfp, Vf2p, W1p, W3, c2, b3)


def kernel(X, emb, W1, b1, V_f, W2, b2, W3, b3):
    B = X.shape[0]
    rows_total = B * INT_FEATURES
    rows_per = rows_total // NUM_WORKERS

    offs = (jnp.arange(INT_FEATURES, dtype=jnp.int32) * VOCAB)[None, :]
    idx = X[:, :INT_FEATURES].astype(jnp.int32) + offs
    idx3d = idx.reshape(NUM_WORKERS, rows_per // CHUNK, CHUNK)

    table16 = emb[:, :, 1:].reshape(INT_FEATURES * VOCAB, EMBED)
    table0 = emb[:, :, 0].reshape(INT_FEATURES * VOCAB)
    gathered, first0 = _sc_gather(table16, table0, idx3d, rows_per)
    E = gathered.reshape(B, INT_FEATURES * EMBED)
    F0 = first0.reshape(B, INT_FEATURES)

    D = INT_FEATURES * EMBED
    W2a = W2[:D]
    S = jnp.tile(jnp.eye(EMBED, dtype=jnp.float32), (INT_FEATURES, 1))
    G = jnp.concatenate(
        [W2a, S, jnp.zeros((D, 256 - HIDDEN - EMBED), jnp.float32)], axis=1)

    c2 = (V_f.reshape(-1) @ W2[D:] + b2).reshape(1, HIDDEN)

    X2 = X[:, INT_FEATURES:INT_FEATURES + FLOAT_FEATURES]
    X2p = jnp.concatenate(
        [X2, jnp.ones((B, 1), jnp.float32), jnp.zeros((B, 2), jnp.float32)],
        axis=1)
    W1p = jnp.concatenate(
        [W1, b1.reshape(1, 1), jnp.zeros((2, 1), jnp.float32)], axis=0)
    Vfp = jnp.concatenate([V_f, jnp.zeros((3, EMBED), jnp.float32)], axis=0)
    Vf2p = Vfp * Vfp

    return _tc_dense(E, F0, X2p, G, S, Vfp, Vf2p, W1p, W3, c2,
                     b3.reshape(1, 1))
